# Initial kernel scaffold; baseline (speedup 1.0000x reference)
import jax
import jax.numpy as jnp
from jax.experimental import pallas as pl


def kernel(true_coords, pred_coords, pred_atoms, true_atoms, pred_charges, true_charges, pred_bonds, true_bonds, batch, bond_aggregation_index, variable_mask, weights):
    raise NotImplementedError("write your pallas kernel here")


# speedup vs baseline: 2.4634x; 2.4634x over previous
import functools

import jax
import jax.numpy as jnp
from jax import lax
from jax.experimental import pallas as pl
from jax.experimental.pallas import tpu as pltpu
from jax.experimental.pallas import tpu_sc as plsc

N = 100000
E = 1600000
B = 1024

N_PAD = 102400
NODE_PAD_BIN = 1025
NBINS = 2048

BS_E = 512
BS_N = 2048

NC, NS = 2, 16
BONDS_PER_TILE = E // NS
CHUNK = 10000
NCHUNK = BONDS_PER_TILE // CHUNK
NODES_PER_TILE = N_PAD // NS


def _bond_ce_body(logits_ref, tgt_ref, out_ref):
    l = logits_ref[...]
    m = jnp.max(l, axis=0, keepdims=True)
    s = jnp.sum(jnp.exp(l - m), axis=0, keepdims=True)
    lse = m + jnp.log(s)
    t = tgt_ref[...].reshape(1, BS_E)
    sel = lax.broadcasted_iota(jnp.int32, (5, BS_E), 0) == t
    picked = jnp.sum(jnp.where(sel, l, 0.0), axis=0, keepdims=True)
    out_ref[...] = (lse - picked)[0]


def _bond_ce(pred_bonds_t, true_bonds):
    return pl.pallas_call(
        _bond_ce_body,
        grid=(E // BS_E,),
        in_specs=[
            pl.BlockSpec((5, BS_E), lambda i: (0, i)),
            pl.BlockSpec((BS_E,), lambda i: (i,)),
        ],
        out_specs=pl.BlockSpec((BS_E,), lambda i: (i,)),
        out_shape=jax.ShapeDtypeStruct((E,), jnp.float32),
    )(pred_bonds_t, true_bonds)


def _node_body(pc_ref, tc_ref, pa_ref, ta_ref, pch_ref, tch_ref, vm_ref,
               regr_ref, atoms_ref, charges_ref):
    vm = vm_ref[...]
    d = pc_ref[...] - tc_ref[...]
    regr = jnp.sum(d * d, axis=0) * (1.0 / 3.0)
    regr_ref[...] = regr * vm

    def ce(logits, tgt, ncls):
        m = jnp.max(logits, axis=0, keepdims=True)
        s = jnp.sum(jnp.exp(logits - m), axis=0, keepdims=True)
        lse = m + jnp.log(s)
        sel = lax.broadcasted_iota(jnp.int32, (ncls, BS_N), 0) == \
            tgt.reshape(1, BS_N)
        picked = jnp.sum(jnp.where(sel, logits, 0.0), axis=0, keepdims=True)
        return (lse - picked)[0]

    atoms_ref[...] = ce(pa_ref[...], ta_ref[...], 16) * vm
    charges_ref[...] = ce(pch_ref[...], tch_ref[...], 6) * vm


def _node_losses(pc_t, tc_t, pa_t, ta, pch_t, tch, vm):
    vec = lambda: pl.BlockSpec((BS_N,), lambda i: (i,))
    mat = lambda r: pl.BlockSpec((r, BS_N), lambda i: (0, i))
    return pl.pallas_call(
        _node_body,
        grid=(N_PAD // BS_N,),
        in_specs=[mat(3), mat(3), mat(16), vec(), mat(6), vec(), vec()],
        out_specs=[vec(), vec(), vec()],
        out_shape=[jax.ShapeDtypeStruct((N_PAD,), jnp.float32)] * 3,
    )(pc_t, tc_t, pa_t, ta, pch_t, tch, vm)


def _sc_body(ce_hbm, bidx_hbm, regr_hbm, atoms_hbm, charges_hbm, batch_hbm,
             w_hbm, zeros_hbm, ones_hbm, bonds_out, ncl_out,
             sp_s, sp_c, sp_g0, sp_g1, sp_g2, sp_gc,
             idx_v, val_v, ones_v, nidx_v, nval_v, s_v, c_v,
             w_v, g_v, gc_v, o_v):
    cid = lax.axis_index("c")
    sid = lax.axis_index("s")

    nb = sid * NODES_PER_TILE
    pltpu.sync_copy(zeros_hbm.at[pl.ds(nb, NODES_PER_TILE)],
                    sp_s.at[pl.ds(nb, NODES_PER_TILE)])
    pltpu.sync_copy(zeros_hbm.at[pl.ds(nb, NODES_PER_TILE)],
                    sp_c.at[pl.ds(nb, NODES_PER_TILE)])

    @pl.when(sid == 0)
    def _zero_bins():
        pltpu.sync_copy(zeros_hbm.at[pl.ds(0, NBINS)], sp_g0)
        pltpu.sync_copy(zeros_hbm.at[pl.ds(0, NBINS)], sp_g1)
        pltpu.sync_copy(zeros_hbm.at[pl.ds(0, NBINS)], sp_g2)
        pltpu.sync_copy(zeros_hbm.at[pl.ds(0, NBINS)], sp_gc)

    pltpu.sync_copy(ones_hbm, ones_v)
    plsc.subcore_barrier()

    @pl.when(cid == 0)
    def _bond_scatter():
        for k in range(NCHUNK):
            off = sid * BONDS_PER_TILE + k * CHUNK
            pltpu.sync_copy(bidx_hbm.at[pl.ds(off, CHUNK)], idx_v)
            pltpu.sync_copy(ce_hbm.at[pl.ds(off, CHUNK)], val_v)
            pltpu.sync_copy(val_v, sp_s.at[idx_v], add=True)
            pltpu.sync_copy(ones_v, sp_c.at[idx_v], add=True)

    @pl.when(cid == 1)
    def _node_scatter():
        base = sid * NODES_PER_TILE
        pltpu.sync_copy(batch_hbm.at[pl.ds(base, NODES_PER_TILE)], nidx_v)
        pltpu.sync_copy(regr_hbm.at[pl.ds(base, NODES_PER_TILE)], nval_v)
        pltpu.sync_copy(nval_v, sp_g0.at[nidx_v], add=True)
        pltpu.sync_copy(atoms_hbm.at[pl.ds(base, NODES_PER_TILE)], nval_v)
        pltpu.sync_copy(nval_v, sp_g1.at[nidx_v], add=True)
        pltpu.sync_copy(charges_hbm.at[pl.ds(base, NODES_PER_TILE)], nval_v)
        pltpu.sync_copy(nval_v, sp_g2.at[nidx_v], add=True)
        pltpu.sync_copy(ones_v.at[pl.ds(0, NODES_PER_TILE)],
                        sp_gc.at[nidx_v], add=True)

    plsc.subcore_barrier()

    @pl.when(cid == 0)
    def _bonds_per_atom():
        base = sid * NODES_PER_TILE
        pltpu.sync_copy(sp_s.at[pl.ds(base, NODES_PER_TILE)], s_v)
        pltpu.sync_copy(sp_c.at[pl.ds(base, NODES_PER_TILE)], c_v)

        def div_step(j, _):
            s = s_v[pl.ds(j * 16, 16)]
            c = c_v[pl.ds(j * 16, 16)]
            s_v[pl.ds(j * 16, 16)] = 0.5 * s / jnp.maximum(c, 1.0)
            return 0

        lax.fori_loop(0, NODES_PER_TILE // 16, div_step, 0)
        pltpu.sync_copy(batch_hbm.at[pl.ds(base, NODES_PER_TILE)], nidx_v)
        pltpu.sync_copy(s_v, sp_g0.at[nidx_v], add=True)
        pltpu.sync_copy(ones_v.at[pl.ds(0, NODES_PER_TILE)],
                        sp_gc.at[nidx_v], add=True)

    plsc.subcore_barrier()

    def weighted_total(g_ref):
        pltpu.sync_copy(g_ref.at[pl.ds(0, B)], g_v)

        def red_step(j, acc):
            s = g_v[pl.ds(j * 16, 16)]
            c = gc_v[pl.ds(j * 16, 16)]
            w = w_v[pl.ds(j * 16, 16)]
            pg = s / jnp.maximum(c, 1.0)
            contrib = jnp.where(pg != pg, 0.0, pg * w)
            return acc + contrib

        return lax.fori_loop(0, B // 16, red_step,
                             jnp.zeros((16,), jnp.float32))

    @pl.when(jnp.logical_and(cid == 0, sid == 0))
    def _fin_bonds():
        pltpu.sync_copy(w_hbm, w_v)
        pltpu.sync_copy(sp_gc.at[pl.ds(0, B)], gc_v)
        o_v[pl.ds(0, 16)] = weighted_total(sp_g0)
        pltpu.sync_copy(o_v.at[pl.ds(0, 16)], bonds_out)

    @pl.when(jnp.logical_and(cid == 1, sid == 0))
    def _fin_nodes():
        pltpu.sync_copy(w_hbm, w_v)
        pltpu.sync_copy(sp_gc.at[pl.ds(0, B)], gc_v)
        o_v[pl.ds(0, 16)] = weighted_total(sp_g0)
        o_v[pl.ds(16, 16)] = weighted_total(sp_g1)
        o_v[pl.ds(32, 16)] = weighted_total(sp_g2)
        pltpu.sync_copy(o_v, ncl_out)


def _sc_reduce(ce, bidx, regr_m, atoms_m, charges_m, batch_p, weights,
               zeros_in, ones_in):
    f32 = jnp.float32
    mesh = plsc.VectorSubcoreMesh(core_axis_name="c", subcore_axis_name="s")
    k = pl.kernel(
        _sc_body,
        out_type=(jax.ShapeDtypeStruct((16,), f32),
                  jax.ShapeDtypeStruct((48,), f32)),
        mesh=mesh,
        scratch_types=[
            pltpu.VMEM_SHARED((N_PAD,), f32),
            pltpu.VMEM_SHARED((N_PAD,), f32),
            pltpu.VMEM_SHARED((NBINS,), f32),
            pltpu.VMEM_SHARED((NBINS,), f32),
            pltpu.VMEM_SHARED((NBINS,), f32),
            pltpu.VMEM_SHARED((NBINS,), f32),
            pltpu.VMEM((CHUNK,), jnp.int32),
            pltpu.VMEM((CHUNK,), f32),
            pltpu.VMEM((CHUNK,), f32),
            pltpu.VMEM((NODES_PER_TILE,), jnp.int32),
            pltpu.VMEM((NODES_PER_TILE,), f32),
            pltpu.VMEM((NODES_PER_TILE,), f32),
            pltpu.VMEM((NODES_PER_TILE,), f32),
            pltpu.VMEM((B,), f32),
            pltpu.VMEM((B,), f32),
            pltpu.VMEM((B,), f32),
            pltpu.VMEM((48,), f32),
        ],
    )
    return k(ce, bidx, regr_m, atoms_m, charges_m, batch_p, weights,
             zeros_in, ones_in)


def kernel(true_coords, pred_coords, pred_atoms, true_atoms, pred_charges,
           true_charges, pred_bonds, true_bonds, batch,
           bond_aggregation_index, variable_mask, weights):
    f32 = jnp.float32
    P = N_PAD - N
    padv = lambda x, v=0: jnp.pad(x, (0, P), constant_values=v)
    padm = lambda x: jnp.pad(x, ((0, 0), (0, P)))

    pbt = pred_bonds.T.astype(f32)
    tb = true_bonds.astype(jnp.int32)
    pc_t = padm(pred_coords.T.astype(f32))
    tc_t = padm(true_coords.T.astype(f32))
    pa_t = padm(pred_atoms.T.astype(f32))
    pch_t = padm(pred_charges.T.astype(f32))
    ta = padv(true_atoms.astype(jnp.int32))
    tch = padv(true_charges.astype(jnp.int32))
    vm = padv(variable_mask.astype(f32))
    batch_p = padv(batch.astype(jnp.int32), NODE_PAD_BIN)
    bidx = bond_aggregation_index.astype(jnp.int32)

    ce = _bond_ce(pbt, tb)
    regr_m, atoms_m, charges_m = _node_losses(pc_t, tc_t, pa_t, ta, pch_t,
                                              tch, vm)

    zeros_in = jnp.zeros((N_PAD,), f32)
    ones_in = jnp.ones((CHUNK,), f32)
    bonds16, ncl48 = _sc_reduce(ce, bidx, regr_m, atoms_m, charges_m,
                                batch_p, weights.astype(f32), zeros_in,
                                ones_in)
    return jnp.stack([jnp.sum(ncl48[0:16]), jnp.sum(ncl48[16:32]),
                      jnp.sum(ncl48[32:48]), jnp.sum(bonds16)])


# speedup vs baseline: 3.0949x; 1.2564x over previous
import functools

import jax
import jax.numpy as jnp
from jax import lax
from jax.experimental import pallas as pl
from jax.experimental.pallas import tpu as pltpu
from jax.experimental.pallas import tpu_sc as plsc

N = 100000
E = 1600000
B = 1024

N_PAD = 102400
NODE_PAD_BIN = 1025
NBINS = 2048

BS_E = 512
BS_N = 2048

NC, NS = 2, 16
BONDS_PER_TILE = E // NS
CHUNK = 10000
NCHUNK = BONDS_PER_TILE // CHUNK
NODES_PER_TILE = N_PAD // NS


GR = 125
RB = 100


def _bond_ce_body(logits_ref, tgt_ref, out_ref, s_scr, st_scr):
    @pl.when(pl.program_id(0) == 0)
    def _build_sel():
        i0 = lax.broadcasted_iota(jnp.int32, (640, 128), 0)
        i1 = lax.broadcasted_iota(jnp.int32, (640, 128), 1)
        d = i0 - 5 * i1
        s_scr[...] = ((d >= 0) & (d < 5)).astype(jnp.float32)
        j0 = lax.broadcasted_iota(jnp.int32, (128, 640), 0)
        j1 = lax.broadcasted_iota(jnp.int32, (128, 640), 1)
        d2 = j1 - 5 * j0
        st_scr[...] = ((d2 >= 0) & (d2 < 5)).astype(jnp.float32)

    dims = (((1,), (0,)), ((), ()))
    l = logits_ref[0]
    se = lax.dot_general(jnp.exp(l), s_scr[...], dims,
                         preferred_element_type=jnp.float32)
    tf = tgt_ref[0].astype(jnp.float32)
    te = lax.dot_general(tf, st_scr[...], dims,
                         preferred_element_type=jnp.float32)
    jf = lax.broadcasted_iota(jnp.int32, (1, 640), 1).astype(jnp.float32)
    cls = jf - 5.0 * jnp.floor(jf * 0.2)
    picked = lax.dot_general(jnp.where(cls == te, l, 0.0), s_scr[...], dims,
                             preferred_element_type=jnp.float32)
    out_ref[0] = jnp.log(se) - picked


def _bond_ce(pred_bonds_r, true_bonds_r):
    return pl.pallas_call(
        _bond_ce_body,
        grid=(GR,),
        in_specs=[
            pl.BlockSpec((1, RB, 640), lambda i: (i, 0, 0)),
            pl.BlockSpec((1, RB, 128), lambda i: (i, 0, 0)),
        ],
        out_specs=pl.BlockSpec((1, RB, 128), lambda i: (i, 0, 0)),
        out_shape=jax.ShapeDtypeStruct((GR, RB, 128), jnp.float32),
        scratch_shapes=[
            pltpu.VMEM((640, 128), jnp.float32),
            pltpu.VMEM((128, 640), jnp.float32),
        ],
    )(pred_bonds_r, true_bonds_r)


def _node_body(pc_ref, tc_ref, pa_ref, ta_ref, pch_ref, tch_ref, vm_ref,
               regr_ref, atoms_ref, charges_ref):
    vm = vm_ref[...]
    d = pc_ref[...] - tc_ref[...]
    regr = jnp.sum(d * d, axis=0) * (1.0 / 3.0)
    regr_ref[...] = regr * vm

    def ce(logits, tgt, ncls):
        m = jnp.max(logits, axis=0, keepdims=True)
        s = jnp.sum(jnp.exp(logits - m), axis=0, keepdims=True)
        lse = m + jnp.log(s)
        sel = lax.broadcasted_iota(jnp.int32, (ncls, BS_N), 0) == \
            tgt.reshape(1, BS_N)
        picked = jnp.sum(jnp.where(sel, logits, 0.0), axis=0, keepdims=True)
        return (lse - picked)[0]

    atoms_ref[...] = ce(pa_ref[...], ta_ref[...], 16) * vm
    charges_ref[...] = ce(pch_ref[...], tch_ref[...], 6) * vm


def _node_losses(pc_t, tc_t, pa_t, ta, pch_t, tch, vm):
    vec = lambda: pl.BlockSpec((BS_N,), lambda i: (i,))
    mat = lambda r: pl.BlockSpec((r, BS_N), lambda i: (0, i))
    return pl.pallas_call(
        _node_body,
        grid=(N_PAD // BS_N,),
        in_specs=[mat(3), mat(3), mat(16), vec(), mat(6), vec(), vec()],
        out_specs=[vec(), vec(), vec()],
        out_shape=[jax.ShapeDtypeStruct((N_PAD,), jnp.float32)] * 3,
    )(pc_t, tc_t, pa_t, ta, pch_t, tch, vm)


def _sc_body(ce_hbm, bidx_hbm, regr_hbm, atoms_hbm, charges_hbm, batch_hbm,
             w_hbm, zeros_hbm, ones_hbm, bonds_out, ncl_out,
             sp_s, sp_c, sp_g0, sp_g1, sp_g2, sp_gc,
             idx_v, val_v, ones_v, nidx_v, nval_v, s_v, c_v,
             w_v, g_v, gc_v, o_v):
    cid = lax.axis_index("c")
    sid = lax.axis_index("s")

    nb = sid * NODES_PER_TILE
    pltpu.sync_copy(zeros_hbm.at[pl.ds(nb, NODES_PER_TILE)],
                    sp_s.at[pl.ds(nb, NODES_PER_TILE)])
    pltpu.sync_copy(zeros_hbm.at[pl.ds(nb, NODES_PER_TILE)],
                    sp_c.at[pl.ds(nb, NODES_PER_TILE)])

    @pl.when(sid == 0)
    def _zero_bins():
        pltpu.sync_copy(zeros_hbm.at[pl.ds(0, NBINS)], sp_g0)
        pltpu.sync_copy(zeros_hbm.at[pl.ds(0, NBINS)], sp_g1)
        pltpu.sync_copy(zeros_hbm.at[pl.ds(0, NBINS)], sp_g2)
        pltpu.sync_copy(zeros_hbm.at[pl.ds(0, NBINS)], sp_gc)

    pltpu.sync_copy(ones_hbm, ones_v)
    plsc.subcore_barrier()

    @pl.when(cid == 0)
    def _bond_scatter():
        for k in range(NCHUNK):
            off = sid * BONDS_PER_TILE + k * CHUNK
            pltpu.sync_copy(bidx_hbm.at[pl.ds(off, CHUNK)], idx_v)
            pltpu.sync_copy(ce_hbm.at[pl.ds(off, CHUNK)], val_v)
            pltpu.sync_copy(val_v, sp_s.at[idx_v], add=True)
            pltpu.sync_copy(ones_v, sp_c.at[idx_v], add=True)

    @pl.when(cid == 1)
    def _node_scatter():
        base = sid * NODES_PER_TILE
        pltpu.sync_copy(batch_hbm.at[pl.ds(base, NODES_PER_TILE)], nidx_v)
        pltpu.sync_copy(regr_hbm.at[pl.ds(base, NODES_PER_TILE)], nval_v)
        pltpu.sync_copy(nval_v, sp_g0.at[nidx_v], add=True)
        pltpu.sync_copy(atoms_hbm.at[pl.ds(base, NODES_PER_TILE)], nval_v)
        pltpu.sync_copy(nval_v, sp_g1.at[nidx_v], add=True)
        pltpu.sync_copy(charges_hbm.at[pl.ds(base, NODES_PER_TILE)], nval_v)
        pltpu.sync_copy(nval_v, sp_g2.at[nidx_v], add=True)
        pltpu.sync_copy(ones_v.at[pl.ds(0, NODES_PER_TILE)],
                        sp_gc.at[nidx_v], add=True)

    plsc.subcore_barrier()

    @pl.when(cid == 0)
    def _bonds_per_atom():
        base = sid * NODES_PER_TILE
        pltpu.sync_copy(sp_s.at[pl.ds(base, NODES_PER_TILE)], s_v)
        pltpu.sync_copy(sp_c.at[pl.ds(base, NODES_PER_TILE)], c_v)

        def div_step(j, _):
            s = s_v[pl.ds(j * 16, 16)]
            c = c_v[pl.ds(j * 16, 16)]
            s_v[pl.ds(j * 16, 16)] = 0.5 * s / jnp.maximum(c, 1.0)
            return 0

        lax.fori_loop(0, NODES_PER_TILE // 16, div_step, 0)
        pltpu.sync_copy(batch_hbm.at[pl.ds(base, NODES_PER_TILE)], nidx_v)
        pltpu.sync_copy(s_v, sp_g0.at[nidx_v], add=True)
        pltpu.sync_copy(ones_v.at[pl.ds(0, NODES_PER_TILE)],
                        sp_gc.at[nidx_v], add=True)

    plsc.subcore_barrier()

    def weighted_total(g_ref):
        pltpu.sync_copy(g_ref.at[pl.ds(0, B)], g_v)

        def red_step(j, acc):
            s = g_v[pl.ds(j * 16, 16)]
            c = gc_v[pl.ds(j * 16, 16)]
            w = w_v[pl.ds(j * 16, 16)]
            pg = s / jnp.maximum(c, 1.0)
            contrib = jnp.where(pg != pg, 0.0, pg * w)
            return acc + contrib

        return lax.fori_loop(0, B // 16, red_step,
                             jnp.zeros((16,), jnp.float32))

    @pl.when(jnp.logical_and(cid == 0, sid == 0))
    def _fin_bonds():
        pltpu.sync_copy(w_hbm, w_v)
        pltpu.sync_copy(sp_gc.at[pl.ds(0, B)], gc_v)
        o_v[pl.ds(0, 16)] = weighted_total(sp_g0)
        pltpu.sync_copy(o_v.at[pl.ds(0, 16)], bonds_out)

    @pl.when(jnp.logical_and(cid == 1, sid == 0))
    def _fin_nodes():
        pltpu.sync_copy(w_hbm, w_v)
        pltpu.sync_copy(sp_gc.at[pl.ds(0, B)], gc_v)
        o_v[pl.ds(0, 16)] = weighted_total(sp_g0)
        o_v[pl.ds(16, 16)] = weighted_total(sp_g1)
        o_v[pl.ds(32, 16)] = weighted_total(sp_g2)
        pltpu.sync_copy(o_v, ncl_out)


def _sc_reduce(ce, bidx, regr_m, atoms_m, charges_m, batch_p, weights,
               zeros_in, ones_in):
    f32 = jnp.float32
    mesh = plsc.VectorSubcoreMesh(core_axis_name="c", subcore_axis_name="s")
    k = pl.kernel(
        _sc_body,
        out_type=(jax.ShapeDtypeStruct((16,), f32),
                  jax.ShapeDtypeStruct((48,), f32)),
        mesh=mesh,
        scratch_types=[
            pltpu.VMEM_SHARED((N_PAD,), f32),
            pltpu.VMEM_SHARED((N_PAD,), f32),
            pltpu.VMEM_SHARED((NBINS,), f32),
            pltpu.VMEM_SHARED((NBINS,), f32),
            pltpu.VMEM_SHARED((NBINS,), f32),
            pltpu.VMEM_SHARED((NBINS,), f32),
            pltpu.VMEM((CHUNK,), jnp.int32),
            pltpu.VMEM((CHUNK,), f32),
            pltpu.VMEM((CHUNK,), f32),
            pltpu.VMEM((NODES_PER_TILE,), jnp.int32),
            pltpu.VMEM((NODES_PER_TILE,), f32),
            pltpu.VMEM((NODES_PER_TILE,), f32),
            pltpu.VMEM((NODES_PER_TILE,), f32),
            pltpu.VMEM((B,), f32),
            pltpu.VMEM((B,), f32),
            pltpu.VMEM((B,), f32),
            pltpu.VMEM((48,), f32),
        ],
    )
    return k(ce, bidx, regr_m, atoms_m, charges_m, batch_p, weights,
             zeros_in, ones_in)


def kernel(true_coords, pred_coords, pred_atoms, true_atoms, pred_charges,
           true_charges, pred_bonds, true_bonds, batch,
           bond_aggregation_index, variable_mask, weights):
    f32 = jnp.float32
    P = N_PAD - N
    padv = lambda x, v=0: jnp.pad(x, (0, P), constant_values=v)
    padm = lambda x: jnp.pad(x, ((0, 0), (0, P)))

    pbr = pred_bonds.astype(f32).reshape(GR, RB, 640)
    tbr = true_bonds.astype(jnp.int32).reshape(GR, RB, 128)
    pc_t = padm(pred_coords.T.astype(f32))
    tc_t = padm(true_coords.T.astype(f32))
    pa_t = padm(pred_atoms.T.astype(f32))
    pch_t = padm(pred_charges.T.astype(f32))
    ta = padv(true_atoms.astype(jnp.int32))
    tch = padv(true_charges.astype(jnp.int32))
    vm = padv(variable_mask.astype(f32))
    batch_p = padv(batch.astype(jnp.int32), NODE_PAD_BIN)
    bidx = bond_aggregation_index.astype(jnp.int32)

    ce = _bond_ce(pbr, tbr).reshape(E)
    regr_m, atoms_m, charges_m = _node_losses(pc_t, tc_t, pa_t, ta, pch_t,
                                              tch, vm)

    zeros_in = jnp.zeros((N_PAD,), f32)
    ones_in = jnp.ones((CHUNK,), f32)
    bonds16, ncl48 = _sc_reduce(ce, bidx, regr_m, atoms_m, charges_m,
                                batch_p, weights.astype(f32), zeros_in,
                                ones_in)
    return jnp.stack([jnp.sum(ncl48[0:16]), jnp.sum(ncl48[16:32]),
                      jnp.sum(ncl48[32:48]), jnp.sum(bonds16)])


# speedup vs baseline: 9.9607x; 3.2185x over previous
import jax
import jax.numpy as jnp
from jax import lax
from jax.experimental import pallas as pl
from jax.experimental.pallas import tpu as pltpu
from jax.experimental.pallas import tpu_sc as plsc

N = 100000
E = 1600000
B = 1024

N_PAD = 102400
NODE_PAD_BIN = 1025
NBINS = 2048

E_PAD = 1605632
BS_CE = 16384
BS_N = 2048

NC, NS = 2, 16
BONDS_PER_TILE = E // NS
CHUNK = 10000
NCHUNK_B = BONDS_PER_TILE // CHUNK
NODES_PER_TILE = N_PAD // NS


def _bond_ce_body(l0, l1, l2, l3, l4, tgt_ref, out_ref):
    cols = [l0[...], l1[...], l2[...], l3[...], l4[...]]
    t = tgt_ref[...]
    m = cols[0]
    for c in cols[1:]:
        m = jnp.maximum(m, c)
    se = jnp.zeros_like(m)
    picked = jnp.zeros_like(m)
    for ci, c in enumerate(cols):
        se = se + jnp.exp(c - m)
        picked = picked + jnp.where(t == ci, c, 0.0)
    out_ref[...] = m + jnp.log(se) - picked


def _bond_ce(cols, tgt):
    v = lambda: pl.BlockSpec((BS_CE,), lambda i: (i,))
    return pl.pallas_call(
        _bond_ce_body,
        grid=(E_PAD // BS_CE,),
        in_specs=[v() for _ in range(6)],
        out_specs=v(),
        out_shape=jax.ShapeDtypeStruct((E_PAD,), jnp.float32),
    )(*cols, tgt)


def _node_body(pc_ref, tc_ref, pa_ref, ta_ref, pch_ref, tch_ref, vm_ref,
               regr_ref, atoms_ref, charges_ref):
    vm = vm_ref[...]
    d = pc_ref[...] - tc_ref[...]
    regr = jnp.sum(d * d, axis=0) * (1.0 / 3.0)
    regr_ref[...] = regr * vm

    def ce(logits, tgt, ncls):
        m = jnp.max(logits, axis=0, keepdims=True)
        s = jnp.sum(jnp.exp(logits - m), axis=0, keepdims=True)
        lse = m + jnp.log(s)
        sel = lax.broadcasted_iota(jnp.int32, (ncls, BS_N), 0) == \
            tgt.reshape(1, BS_N)
        picked = jnp.sum(jnp.where(sel, logits, 0.0), axis=0, keepdims=True)
        return (lse - picked)[0]

    atoms_ref[...] = ce(pa_ref[...], ta_ref[...], 16) * vm
    charges_ref[...] = ce(pch_ref[...], tch_ref[...], 6) * vm


def _node_losses(pc_t, tc_t, pa_t, ta, pch_t, tch, vm):
    vec = lambda: pl.BlockSpec((BS_N,), lambda i: (i,))
    mat = lambda r: pl.BlockSpec((r, BS_N), lambda i: (0, i))
    return pl.pallas_call(
        _node_body,
        grid=(N_PAD // BS_N,),
        in_specs=[mat(3), mat(3), mat(16), vec(), mat(6), vec(), vec()],
        out_specs=[vec(), vec(), vec()],
        out_shape=[jax.ShapeDtypeStruct((N_PAD,), jnp.float32)] * 3,
    )(pc_t, tc_t, pa_t, ta, pch_t, tch, vm)


def _sc_body(ce_hbm, bidx_hbm, regr_hbm, atoms_hbm, charges_hbm, batch_hbm,
             w_hbm, zeros_hbm, ones_hbm, bonds_out, ncl_out,
             sp_s, sp_c, sp_g0, sp_g1, sp_g2, sp_gc,
             idx_v, val_v, ones_v, nidx_v, nval_v, s_v, c_v,
             w_v, g_v, gc_v, o_v):
    cid = lax.axis_index("c")
    sid = lax.axis_index("s")

    nb = sid * NODES_PER_TILE
    pltpu.sync_copy(zeros_hbm.at[pl.ds(nb, NODES_PER_TILE)],
                    sp_s.at[pl.ds(nb, NODES_PER_TILE)])
    pltpu.sync_copy(zeros_hbm.at[pl.ds(nb, NODES_PER_TILE)],
                    sp_c.at[pl.ds(nb, NODES_PER_TILE)])

    @pl.when(sid == 0)
    def _zero_bins():
        pltpu.sync_copy(zeros_hbm.at[pl.ds(0, NBINS)], sp_g0)
        pltpu.sync_copy(zeros_hbm.at[pl.ds(0, NBINS)], sp_g1)
        pltpu.sync_copy(zeros_hbm.at[pl.ds(0, NBINS)], sp_g2)
        pltpu.sync_copy(zeros_hbm.at[pl.ds(0, NBINS)], sp_gc)

    pltpu.sync_copy(ones_hbm, ones_v)
    plsc.subcore_barrier()

    @pl.when(cid == 0)
    def _bond_scatter():
        for k in range(NCHUNK_B):
            off = sid * BONDS_PER_TILE + k * CHUNK
            pltpu.sync_copy(bidx_hbm.at[pl.ds(off, CHUNK)], idx_v)
            pltpu.sync_copy(ce_hbm.at[pl.ds(off, CHUNK)], val_v)
            pltpu.sync_copy(val_v, sp_s.at[idx_v], add=True)
            pltpu.sync_copy(ones_v, sp_c.at[idx_v], add=True)

    @pl.when(cid == 1)
    def _node_scatter():
        base = sid * NODES_PER_TILE
        pltpu.sync_copy(batch_hbm.at[pl.ds(base, NODES_PER_TILE)], nidx_v)
        pltpu.sync_copy(regr_hbm.at[pl.ds(base, NODES_PER_TILE)], nval_v)
        pltpu.sync_copy(nval_v, sp_g0.at[nidx_v], add=True)
        pltpu.sync_copy(atoms_hbm.at[pl.ds(base, NODES_PER_TILE)], nval_v)
        pltpu.sync_copy(nval_v, sp_g1.at[nidx_v], add=True)
        pltpu.sync_copy(charges_hbm.at[pl.ds(base, NODES_PER_TILE)], nval_v)
        pltpu.sync_copy(nval_v, sp_g2.at[nidx_v], add=True)
        pltpu.sync_copy(ones_v.at[pl.ds(0, NODES_PER_TILE)],
                        sp_gc.at[nidx_v], add=True)

    plsc.subcore_barrier()

    @pl.when(cid == 0)
    def _bonds_per_atom():
        base = sid * NODES_PER_TILE
        pltpu.sync_copy(sp_s.at[pl.ds(base, NODES_PER_TILE)], s_v)
        pltpu.sync_copy(sp_c.at[pl.ds(base, NODES_PER_TILE)], c_v)

        def div_step(j, _):
            s = s_v[pl.ds(j * 16, 16)]
            c = c_v[pl.ds(j * 16, 16)]
            s_v[pl.ds(j * 16, 16)] = 0.5 * s / jnp.maximum(c, 1.0)
            return 0

        lax.fori_loop(0, NODES_PER_TILE // 16, div_step, 0)
        pltpu.sync_copy(batch_hbm.at[pl.ds(base, NODES_PER_TILE)], nidx_v)
        pltpu.sync_copy(s_v, sp_g0.at[nidx_v], add=True)
        pltpu.sync_copy(ones_v.at[pl.ds(0, NODES_PER_TILE)],
                        sp_gc.at[nidx_v], add=True)

    plsc.subcore_barrier()

    def weighted_total(g_ref):
        pltpu.sync_copy(g_ref.at[pl.ds(0, B)], g_v)

        def red_step(j, acc):
            s = g_v[pl.ds(j * 16, 16)]
            c = gc_v[pl.ds(j * 16, 16)]
            w = w_v[pl.ds(j * 16, 16)]
            pg = s / jnp.maximum(c, 1.0)
            contrib = jnp.where(pg != pg, 0.0, pg * w)
            return acc + contrib

        return lax.fori_loop(0, B // 16, red_step,
                             jnp.zeros((16,), jnp.float32))

    @pl.when(jnp.logical_and(cid == 0, sid == 0))
    def _fin_bonds():
        pltpu.sync_copy(w_hbm, w_v)
        pltpu.sync_copy(sp_gc.at[pl.ds(0, B)], gc_v)
        o_v[pl.ds(0, 16)] = weighted_total(sp_g0)
        pltpu.sync_copy(o_v.at[pl.ds(0, 16)], bonds_out)

    @pl.when(jnp.logical_and(cid == 1, sid == 0))
    def _fin_nodes():
        pltpu.sync_copy(w_hbm, w_v)
        pltpu.sync_copy(sp_gc.at[pl.ds(0, B)], gc_v)
        o_v[pl.ds(0, 16)] = weighted_total(sp_g0)
        o_v[pl.ds(16, 16)] = weighted_total(sp_g1)
        o_v[pl.ds(32, 16)] = weighted_total(sp_g2)
        pltpu.sync_copy(o_v, ncl_out)


def _sc_reduce(ce, bidx, regr_m, atoms_m, charges_m, batch_p, weights,
               zeros_in, ones_in):
    f32 = jnp.float32
    mesh = plsc.VectorSubcoreMesh(core_axis_name="c", subcore_axis_name="s")
    k = pl.kernel(
        _sc_body,
        out_type=(jax.ShapeDtypeStruct((16,), f32),
                  jax.ShapeDtypeStruct((48,), f32)),
        mesh=mesh,
        scratch_types=[
            pltpu.VMEM_SHARED((N_PAD,), f32),
            pltpu.VMEM_SHARED((N_PAD,), f32),
            pltpu.VMEM_SHARED((NBINS,), f32),
            pltpu.VMEM_SHARED((NBINS,), f32),
            pltpu.VMEM_SHARED((NBINS,), f32),
            pltpu.VMEM_SHARED((NBINS,), f32),
            pltpu.VMEM((CHUNK,), jnp.int32),
            pltpu.VMEM((CHUNK,), f32),
            pltpu.VMEM((CHUNK,), f32),
            pltpu.VMEM((NODES_PER_TILE,), jnp.int32),
            pltpu.VMEM((NODES_PER_TILE,), f32),
            pltpu.VMEM((NODES_PER_TILE,), f32),
            pltpu.VMEM((NODES_PER_TILE,), f32),
            pltpu.VMEM((B,), f32),
            pltpu.VMEM((B,), f32),
            pltpu.VMEM((B,), f32),
            pltpu.VMEM((48,), f32),
        ],
    )
    return k(ce, bidx, regr_m, atoms_m, charges_m, batch_p, weights,
             zeros_in, ones_in)


def kernel(true_coords, pred_coords, pred_atoms, true_atoms, pred_charges,
           true_charges, pred_bonds, true_bonds, batch,
           bond_aggregation_index, variable_mask, weights):
    f32 = jnp.float32
    P = N_PAD - N
    padv = lambda x, v=0: jnp.pad(x, (0, P), constant_values=v)
    padm = lambda x: jnp.pad(x, ((0, 0), (0, P)))

    pc_t = padm(pred_coords.T.astype(f32))
    tc_t = padm(true_coords.T.astype(f32))
    pa_t = padm(pred_atoms.T.astype(f32))
    pch_t = padm(pred_charges.T.astype(f32))
    ta = padv(true_atoms.astype(jnp.int32))
    tch = padv(true_charges.astype(jnp.int32))
    vm = padv(variable_mask.astype(f32))
    batch_p = padv(batch.astype(jnp.int32), NODE_PAD_BIN)
    bidx = bond_aggregation_index.astype(jnp.int32)

    cols = [jnp.pad(pred_bonds[:, c].astype(f32), (0, E_PAD - E))
            for c in range(5)]
    tbp = jnp.pad(true_bonds.astype(jnp.int32), (0, E_PAD - E))
    ce = _bond_ce(cols, tbp)
    regr_m, atoms_m, charges_m = _node_losses(pc_t, tc_t, pa_t, ta, pch_t,
                                              tch, vm)

    zeros_in = jnp.zeros((N_PAD,), f32)
    ones_in = jnp.ones((CHUNK,), f32)
    bonds16, ncl48 = _sc_reduce(ce, bidx, regr_m, atoms_m, charges_m,
                                batch_p, weights.astype(f32), zeros_in,
                                ones_in)
    return jnp.stack([jnp.sum(ncl48[0:16]), jnp.sum(ncl48[16:32]),
                      jnp.sum(ncl48[32:48]), jnp.sum(bonds16)])


# speedup vs baseline: 10.3416x; 1.0382x over previous
import jax
import jax.numpy as jnp
from jax import lax
from jax.experimental import pallas as pl
from jax.experimental.pallas import tpu as pltpu
from jax.experimental.pallas import tpu_sc as plsc

N = 100000
E = 1600000
B = 1024

N_PAD = 102400
NODE_PAD_BIN = 1025
NBINS = 2048

E_PAD = 1605632
BS_CE = 16384
BS_N = 2048

NC, NS = 2, 16
BONDS_PER_TILE = E // NS
CHUNK = 10000
NCHUNK_B = BONDS_PER_TILE // CHUNK
NODES_PER_TILE = N_PAD // NS


def _bond_ce_body(l0, l1, l2, l3, l4, tgt_ref, out_ref):
    cols = [l0[...], l1[...], l2[...], l3[...], l4[...]]
    t = tgt_ref[...]
    m = cols[0]
    for c in cols[1:]:
        m = jnp.maximum(m, c)
    se = jnp.zeros_like(m)
    picked = jnp.zeros_like(m)
    for ci, c in enumerate(cols):
        se = se + jnp.exp(c - m)
        picked = picked + jnp.where(t == ci, c, 0.0)
    out_ref[...] = m + jnp.log(se) - picked


def _bond_ce(cols, tgt):
    v = lambda: pl.BlockSpec((BS_CE,), lambda i: (i,))
    return pl.pallas_call(
        _bond_ce_body,
        grid=(E_PAD // BS_CE,),
        in_specs=[v() for _ in range(6)],
        out_specs=v(),
        out_shape=jax.ShapeDtypeStruct((E_PAD,), jnp.float32),
    )(*cols, tgt)


def _node_body(pc_ref, tc_ref, pa_ref, ta_ref, pch_ref, tch_ref, vm_ref,
               regr_ref, atoms_ref, charges_ref):
    vm = vm_ref[...]
    d = pc_ref[...] - tc_ref[...]
    regr = jnp.sum(d * d, axis=0) * (1.0 / 3.0)
    regr_ref[...] = regr * vm

    def ce(logits, tgt, ncls):
        m = jnp.max(logits, axis=0, keepdims=True)
        s = jnp.sum(jnp.exp(logits - m), axis=0, keepdims=True)
        lse = m + jnp.log(s)
        sel = lax.broadcasted_iota(jnp.int32, (ncls, BS_N), 0) == \
            tgt.reshape(1, BS_N)
        picked = jnp.sum(jnp.where(sel, logits, 0.0), axis=0, keepdims=True)
        return (lse - picked)[0]

    atoms_ref[...] = ce(pa_ref[...], ta_ref[...], 16) * vm
    charges_ref[...] = ce(pch_ref[...], tch_ref[...], 6) * vm


def _node_losses(pc_t, tc_t, pa_t, ta, pch_t, tch, vm):
    vec = lambda: pl.BlockSpec((BS_N,), lambda i: (i,))
    mat = lambda r: pl.BlockSpec((r, BS_N), lambda i: (0, i))
    return pl.pallas_call(
        _node_body,
        grid=(N_PAD // BS_N,),
        in_specs=[mat(3), mat(3), mat(16), vec(), mat(6), vec(), vec()],
        out_specs=[vec(), vec(), vec()],
        out_shape=[jax.ShapeDtypeStruct((N_PAD,), jnp.float32)] * 3,
    )(pc_t, tc_t, pa_t, ta, pch_t, tch, vm)


def _sc_body(ce_hbm, bidx_hbm, regr_hbm, atoms_hbm, charges_hbm, batch_hbm,
             w_hbm, zeros_hbm, ones_hbm, bonds_out, ncl_out,
             sp_s, sp_c, sp_g0, sp_g1, sp_g2, sp_gc,
             idx_a, idx_b, val_a, val_b, ones_v, nidx_v, nval_v, s_v, c_v,
             w_v, g_v, gc_v, o_v, semi0, semi1, semv0, semv1):
    cid = lax.axis_index("c")
    sid = lax.axis_index("s")

    nb = sid * NODES_PER_TILE
    pltpu.sync_copy(zeros_hbm.at[pl.ds(nb, NODES_PER_TILE)],
                    sp_s.at[pl.ds(nb, NODES_PER_TILE)])
    pltpu.sync_copy(zeros_hbm.at[pl.ds(nb, NODES_PER_TILE)],
                    sp_c.at[pl.ds(nb, NODES_PER_TILE)])

    @pl.when(sid == 0)
    def _zero_bins():
        pltpu.sync_copy(zeros_hbm.at[pl.ds(0, NBINS)], sp_g0)
        pltpu.sync_copy(zeros_hbm.at[pl.ds(0, NBINS)], sp_g1)
        pltpu.sync_copy(zeros_hbm.at[pl.ds(0, NBINS)], sp_g2)
        pltpu.sync_copy(zeros_hbm.at[pl.ds(0, NBINS)], sp_gc)

    pltpu.sync_copy(ones_hbm, ones_v)
    plsc.subcore_barrier()

    @pl.when(cid == 0)
    def _bond_scatter():
        semi = (semi0, semi1)
        semv = (semv0, semv1)
        idxb = (idx_a, idx_b)
        valb = (val_a, val_b)
        descs = [None, None]

        def start(k):
            b = k % 2
            off = sid * BONDS_PER_TILE + k * CHUNK
            descs[b] = (
                pltpu.async_copy(bidx_hbm.at[pl.ds(off, CHUNK)],
                                 idxb[b], semi[b]),
                pltpu.async_copy(ce_hbm.at[pl.ds(off, CHUNK)],
                                 valb[b], semv[b]),
            )

        start(0)
        for k in range(NCHUNK_B):
            b = k % 2
            descs[b][0].wait()
            descs[b][1].wait()
            if k + 1 < NCHUNK_B:
                start(k + 1)
            pltpu.sync_copy(valb[b], sp_s.at[idxb[b]], add=True)
            pltpu.sync_copy(ones_v, sp_c.at[idxb[b]], add=True)

    @pl.when(cid == 1)
    def _node_scatter():
        base = sid * NODES_PER_TILE
        pltpu.sync_copy(batch_hbm.at[pl.ds(base, NODES_PER_TILE)], nidx_v)
        pltpu.sync_copy(regr_hbm.at[pl.ds(base, NODES_PER_TILE)], nval_v)
        pltpu.sync_copy(nval_v, sp_g0.at[nidx_v], add=True)
        pltpu.sync_copy(atoms_hbm.at[pl.ds(base, NODES_PER_TILE)], nval_v)
        pltpu.sync_copy(nval_v, sp_g1.at[nidx_v], add=True)
        pltpu.sync_copy(charges_hbm.at[pl.ds(base, NODES_PER_TILE)], nval_v)
        pltpu.sync_copy(nval_v, sp_g2.at[nidx_v], add=True)
        pltpu.sync_copy(ones_v.at[pl.ds(0, NODES_PER_TILE)],
                        sp_gc.at[nidx_v], add=True)

    plsc.subcore_barrier()

    @pl.when(cid == 0)
    def _bonds_per_atom():
        base = sid * NODES_PER_TILE
        pltpu.sync_copy(sp_s.at[pl.ds(base, NODES_PER_TILE)], s_v)
        pltpu.sync_copy(sp_c.at[pl.ds(base, NODES_PER_TILE)], c_v)

        def div_step(j, _):
            s = s_v[pl.ds(j * 16, 16)]
            c = c_v[pl.ds(j * 16, 16)]
            s_v[pl.ds(j * 16, 16)] = 0.5 * s / jnp.maximum(c, 1.0)
            return 0

        lax.fori_loop(0, NODES_PER_TILE // 16, div_step, 0)
        pltpu.sync_copy(batch_hbm.at[pl.ds(base, NODES_PER_TILE)], nidx_v)
        pltpu.sync_copy(s_v, sp_g0.at[nidx_v], add=True)
        pltpu.sync_copy(ones_v.at[pl.ds(0, NODES_PER_TILE)],
                        sp_gc.at[nidx_v], add=True)

    plsc.subcore_barrier()

    def weighted_total(g_ref):
        pltpu.sync_copy(g_ref.at[pl.ds(0, B)], g_v)

        def red_step(j, acc):
            s = g_v[pl.ds(j * 16, 16)]
            c = gc_v[pl.ds(j * 16, 16)]
            w = w_v[pl.ds(j * 16, 16)]
            pg = s / jnp.maximum(c, 1.0)
            contrib = jnp.where(pg != pg, 0.0, pg * w)
            return acc + contrib

        return lax.fori_loop(0, B // 16, red_step,
                             jnp.zeros((16,), jnp.float32))

    @pl.when(jnp.logical_and(cid == 0, sid == 0))
    def _fin_bonds():
        pltpu.sync_copy(w_hbm, w_v)
        pltpu.sync_copy(sp_gc.at[pl.ds(0, B)], gc_v)
        o_v[pl.ds(0, 16)] = weighted_total(sp_g0)
        pltpu.sync_copy(o_v.at[pl.ds(0, 16)], bonds_out)

    @pl.when(jnp.logical_and(cid == 1, sid == 0))
    def _fin_nodes():
        pltpu.sync_copy(w_hbm, w_v)
        pltpu.sync_copy(sp_gc.at[pl.ds(0, B)], gc_v)
        o_v[pl.ds(0, 16)] = weighted_total(sp_g0)
        o_v[pl.ds(16, 16)] = weighted_total(sp_g1)
        o_v[pl.ds(32, 16)] = weighted_total(sp_g2)
        pltpu.sync_copy(o_v, ncl_out)


def _sc_reduce(ce, bidx, regr_m, atoms_m, charges_m, batch_p, weights,
               zeros_in, ones_in):
    f32 = jnp.float32
    mesh = plsc.VectorSubcoreMesh(core_axis_name="c", subcore_axis_name="s")
    k = pl.kernel(
        _sc_body,
        out_type=(jax.ShapeDtypeStruct((16,), f32),
                  jax.ShapeDtypeStruct((48,), f32)),
        mesh=mesh,
        scratch_types=[
            pltpu.VMEM_SHARED((N_PAD,), f32),
            pltpu.VMEM_SHARED((N_PAD,), f32),
            pltpu.VMEM_SHARED((NBINS,), f32),
            pltpu.VMEM_SHARED((NBINS,), f32),
            pltpu.VMEM_SHARED((NBINS,), f32),
            pltpu.VMEM_SHARED((NBINS,), f32),
            pltpu.VMEM((CHUNK,), jnp.int32),
            pltpu.VMEM((CHUNK,), jnp.int32),
            pltpu.VMEM((CHUNK,), f32),
            pltpu.VMEM((CHUNK,), f32),
            pltpu.VMEM((CHUNK,), f32),
            pltpu.VMEM((NODES_PER_TILE,), jnp.int32),
            pltpu.VMEM((NODES_PER_TILE,), f32),
            pltpu.VMEM((NODES_PER_TILE,), f32),
            pltpu.VMEM((NODES_PER_TILE,), f32),
            pltpu.VMEM((B,), f32),
            pltpu.VMEM((B,), f32),
            pltpu.VMEM((B,), f32),
            pltpu.VMEM((48,), f32),
            pltpu.SemaphoreType.DMA,
            pltpu.SemaphoreType.DMA,
            pltpu.SemaphoreType.DMA,
            pltpu.SemaphoreType.DMA,
        ],
    )
    return k(ce, bidx, regr_m, atoms_m, charges_m, batch_p, weights,
             zeros_in, ones_in)


def kernel(true_coords, pred_coords, pred_atoms, true_atoms, pred_charges,
           true_charges, pred_bonds, true_bonds, batch,
           bond_aggregation_index, variable_mask, weights):
    f32 = jnp.float32
    P = N_PAD - N
    padv = lambda x, v=0: jnp.pad(x, (0, P), constant_values=v)
    padm = lambda x: jnp.pad(x, ((0, 0), (0, P)))

    pc_t = padm(pred_coords.T.astype(f32))
    tc_t = padm(true_coords.T.astype(f32))
    pa_t = padm(pred_atoms.T.astype(f32))
    pch_t = padm(pred_charges.T.astype(f32))
    ta = padv(true_atoms.astype(jnp.int32))
    tch = padv(true_charges.astype(jnp.int32))
    vm = padv(variable_mask.astype(f32))
    batch_p = padv(batch.astype(jnp.int32), NODE_PAD_BIN)
    bidx = bond_aggregation_index.astype(jnp.int32)

    cols = [jnp.pad(pred_bonds[:, c].astype(f32), (0, E_PAD - E))
            for c in range(5)]
    tbp = jnp.pad(true_bonds.astype(jnp.int32), (0, E_PAD - E))
    ce = _bond_ce(cols, tbp)
    regr_m, atoms_m, charges_m = _node_losses(pc_t, tc_t, pa_t, ta, pch_t,
                                              tch, vm)

    zeros_in = jnp.zeros((N_PAD,), f32)
    ones_in = jnp.ones((CHUNK,), f32)
    bonds16, ncl48 = _sc_reduce(ce, bidx, regr_m, atoms_m, charges_m,
                                batch_p, weights.astype(f32), zeros_in,
                                ones_in)
    return jnp.stack([jnp.sum(ncl48[0:16]), jnp.sum(ncl48[16:32]),
                      jnp.sum(ncl48[32:48]), jnp.sum(bonds16)])
